# baseline (speedup 1.0000x reference)
import functools

import jax
import jax.numpy as jnp
from jax import lax
from jax.experimental import pallas as pl
from jax.experimental.pallas import tpu as pltpu
from jax.experimental.pallas import tpu_sc as plsc

B = 16
C = 384
CH = C // 2
HW = 4096
P = 256
NC, NS = 2, 16
CHH = CH // 2
CC = 8
NCHUNK = CHH // CC
L = 16
NG = P // L


def _rsqrt(s):
    i = plsc.bitcast(s, jnp.int32)
    i = jnp.int32(0x5F3759DF) - lax.shift_right_arithmetic(i, 1)
    y = plsc.bitcast(i, jnp.float32)
    half = s * 0.5
    for _ in range(3):
        y = y * (1.5 - half * y * y)
    return y


def _sc_body(x_hbm, pid_hbm, out_hbm, pid_v, buf, out_local, ssq, part,
             fac, shared_ssq, shared_final):
    cid = lax.axis_index("c")
    sid = lax.axis_index("s")
    b = cid * 8 + lax.div(sid, 2)
    half = lax.rem(sid, 2)
    row0 = b * C + half * CHH

    pltpu.sync_copy(pid_hbm, pid_v)

    zeros = jnp.zeros((L,), jnp.float32)
    for g in range(NG):
        ssq[pl.ds(g * L, L)] = zeros

    iota = lax.iota(jnp.int32, L)

    def chan_body(j, carry):
        jv = jnp.full((L,), j, dtype=jnp.int32)
        col = carry
        colv = jnp.full((L,), col, dtype=jnp.int32)
        for g in range(NG):
            hw = pid_v[pl.ds(g * L, L)]
            vals = plsc.load_gather(buf, [jv, hw])
            prow = iota + (g * L)
            plsc.store_scatter(out_local, [prow, colv], vals)
            plsc.addupdate(ssq.at[pl.ds(g * L, L)], vals * vals)
        return col + 1

    for k in range(NCHUNK):
        pltpu.sync_copy(x_hbm.at[pl.ds(row0 + k * CC, CC)], buf)
        lax.fori_loop(0, CC, chan_body, jnp.int32(k * CC))

    pltpu.sync_copy(ssq, shared_ssq.at[sid])
    plsc.subcore_barrier()
    pltpu.sync_copy(shared_ssq.at[sid ^ 1], part)

    for g in range(NG):
        s_tot = ssq[pl.ds(g * L, L)] + part[pl.ds(g * L, L)]
        norm = s_tot * _rsqrt(s_tot)
        fac[pl.ds(g * L, L)] = 1.0 / (norm + 1e-7)

    def scale_body(g, _):
        fv = fac[pl.ds(g * L, L)]
        for l in range(L):
            f = jnp.full((L,), fv[l], dtype=jnp.float32)
            p = g * L + l
            for t in range(CHH // L):
                v = out_local[p, pl.ds(t * L, L)]
                out_local[p, pl.ds(t * L, L)] = v * f
        return 0

    lax.fori_loop(0, NG, scale_body, 0)

    pair0 = sid & ~1
    pair1 = sid | 1
    cstart = half * CHH
    pltpu.sync_copy(out_local.at[pl.ds(0, P // 2)],
                    shared_final.at[pair0, :, pl.ds(cstart, CHH)])
    pltpu.sync_copy(out_local.at[pl.ds(P // 2, P // 2)],
                    shared_final.at[pair1, :, pl.ds(cstart, CHH)])
    plsc.subcore_barrier()
    pltpu.sync_copy(
        shared_final.at[sid],
        out_hbm.at[pl.ds(b * P + half * (P // 2), P // 2)])


@jax.jit
def _run(x2, patch_ids):
    mesh = plsc.VectorSubcoreMesh(
        core_axis_name="c", subcore_axis_name="s",
        num_cores=NC, num_subcores=NS)
    f = pl.kernel(
        _sc_body,
        out_type=jax.ShapeDtypeStruct((B * P, CH), jnp.float32),
        mesh=mesh,
        scratch_types=[
            pltpu.VMEM((P,), jnp.int32),
            pltpu.VMEM((CC, HW), jnp.float32),
            pltpu.VMEM((P, CHH), jnp.float32),
            pltpu.VMEM((P,), jnp.float32),
            pltpu.VMEM((P,), jnp.float32),
            pltpu.VMEM((P,), jnp.float32),
            pltpu.VMEM_SHARED((NS, P), jnp.float32),
            pltpu.VMEM_SHARED((NS, P // 2, CH), jnp.float32),
        ],
        compiler_params=pltpu.CompilerParams(
            use_tc_tiling_on_sc=False, needs_layout_passes=False),
    )
    return f(x2, patch_ids)


def kernel(x, num_patches, patch_ids):
    x2 = x.reshape(B * C, HW)
    out = _run(x2, patch_ids)
    return (out, patch_ids)
